# Initial kernel scaffold; baseline (speedup 1.0000x reference)
import jax
import jax.numpy as jnp
from jax.experimental import pallas as pl


def kernel(x_0, x_1, x_2, edge_index_0, edge_index_1, edge_index_2, layer_edge_index_0, layer_edge_index_1, layer_edge_index_2, W_lg_0, b_lg_0, W_lg_1, b_lg_1, W_lg_2, b_lg_2, W_og_0, b_og_0, W_og_1, b_og_1, W_og_2, b_og_2, W_p1, b_p1, W_p2, b_p2):
    raise NotImplementedError("write your pallas kernel here")


# speedup vs baseline: 25.8905x; 25.8905x over previous
import functools

import jax
import jax.numpy as jnp
from jax import lax
from jax.experimental import pallas as pl
from jax.experimental.pallas import tpu as pltpu
from jax.experimental.pallas import tpu_sc as plsc

N = 50000
F = 32
NC = 2
NS = 16
NW = NC * NS
SUB = 128
KSUB = 4
CH = SUB * KSUB
NPAD = 50048
ZR = NPAD // NS
OR = ZR

_MESH = dict(core_axis_name="c", subcore_axis_name="s")


def _pad_len(m):
    blk = NW * CH
    return ((m + blk - 1) // blk) * blk


def _pad_gather_idx(idx, mpad):
    pad = mpad - idx.shape[0]
    fill = lax.iota(jnp.int32, pad) % 128
    return jnp.concatenate([idx.astype(jnp.int32), fill]).reshape(mpad // SUB, SUB)


def _pad_scatter_idx(idx, mpad):
    pad = mpad - idx.shape[0]
    fill = N + (lax.iota(jnp.int32, pad) % 16)
    return jnp.concatenate([idx.astype(jnp.int32), fill]).reshape(mpad // SUB, SUB)


@functools.lru_cache(maxsize=None)
def _make_prop(mpad):
    nchunks = mpad // (NW * CH)
    rows_per_w = mpad // SUB // NW

    @functools.partial(
        pl.kernel,
        out_type=jax.ShapeDtypeStruct((NC, NPAD, F), jnp.float32),
        mesh=plsc.VectorSubcoreMesh(**_MESH),
        scratch_types=[
            pltpu.VMEM((KSUB, SUB), jnp.int32),
            pltpu.VMEM((KSUB, SUB), jnp.int32),
            pltpu.VMEM((KSUB, SUB, F), jnp.float32),
            pltpu.VMEM_SHARED((NPAD, F), jnp.float32),
            pltpu.SemaphoreType.DMA,
            pltpu.SemaphoreType.DMA,
        ],
        compiler_params=pltpu.CompilerParams(use_tc_tiling_on_sc=False),
    )
    def prop(table, gidx, sidx, zrows, out, gi_v, si_v, rows_v, acc, sem_g, sem_s):
        c = lax.axis_index("c")
        s = lax.axis_index("s")
        w = c * NS + s
        pltpu.sync_copy(zrows, acc.at[pl.ds(s * ZR, ZR)])
        plsc.subcore_barrier()
        base_row = w * rows_per_w

        def chunk(j, carry):
            r0 = base_row + j * KSUB
            pltpu.sync_copy(gidx.at[pl.ds(r0, KSUB)], gi_v)
            pltpu.sync_copy(sidx.at[pl.ds(r0, KSUB)], si_v)
            gets = [
                pltpu.async_copy(table.at[gi_v.at[k]], rows_v.at[k], sem_g)
                for k in range(KSUB)
            ]
            for cp in gets:
                cp.wait()
            puts = [
                pltpu.async_copy(rows_v.at[k], acc.at[si_v.at[k]], sem_s, add=True)
                for k in range(KSUB)
            ]
            for cp in puts:
                cp.wait()
            return carry

        lax.fori_loop(0, nchunks, chunk, 0)
        plsc.subcore_barrier()
        pltpu.sync_copy(acc.at[pl.ds(s * OR, OR)], out.at[c, pl.ds(s * OR, OR)])

    return prop


@functools.lru_cache(maxsize=None)
def _make_deg(mpad):
    rows_per_s = mpad // SUB // NS
    nchunks = rows_per_s // KSUB

    @functools.partial(
        pl.kernel,
        out_type=jax.ShapeDtypeStruct((NC, NPAD, 16), jnp.float32),
        mesh=plsc.VectorSubcoreMesh(**_MESH),
        scratch_types=[
            pltpu.VMEM((KSUB, SUB), jnp.int32),
            pltpu.VMEM((SUB, 16), jnp.float32),
            pltpu.VMEM_SHARED((NPAD, 16), jnp.float32),
            pltpu.SemaphoreType.DMA,
        ],
        compiler_params=pltpu.CompilerParams(use_tc_tiling_on_sc=False),
    )
    def deg(dsts, ones_hbm, zrows, out, si_v, ones_v, acc, sem_s):
        c = lax.axis_index("c")
        s = lax.axis_index("s")
        pltpu.sync_copy(zrows, acc.at[pl.ds(s * ZR, ZR)])
        pltpu.sync_copy(ones_hbm, ones_v)
        plsc.subcore_barrier()
        base_row = s * rows_per_s

        def chunk(j, carry):
            r0 = base_row + j * KSUB
            pltpu.sync_copy(dsts.at[c, pl.ds(r0, KSUB)], si_v)
            puts = [
                pltpu.async_copy(ones_v, acc.at[si_v.at[k]], sem_s, add=True)
                for k in range(KSUB)
            ]
            for cp in puts:
                cp.wait()
            return carry

        lax.fori_loop(0, nchunks, chunk, 0)
        plsc.subcore_barrier()
        pltpu.sync_copy(acc.at[pl.ds(s * OR, OR)], out.at[c, pl.ds(s * OR, OR)])

    return deg


BN = 1000
GRID = N // BN


def _row_spec(width):
    return pl.BlockSpec((BN, width), lambda i: (i, 0))


def _part_spec(width):
    return pl.BlockSpec((NC, BN, width), lambda i: (0, i, 0))


def _full_spec(shape):
    nd = len(shape)
    return pl.BlockSpec(shape, lambda i: (0,) * nd)


def _tc_call(body, in_specs, n_out, out_widths):
    return pl.pallas_call(
        body,
        grid=(GRID,),
        in_specs=in_specs,
        out_specs=tuple(_row_spec(w) for w in out_widths),
        out_shape=tuple(
            jax.ShapeDtypeStruct((N, w), jnp.float32) for w in out_widths
        ),
    )


def _tc_prep_body(cnt, x1, x2, w1, w2, d1o, d2o, g1o, g2o):
    c = cnt[...]
    d1 = lax.rsqrt(c[0, :, 0:1] + 1.0)
    d2 = lax.rsqrt(c[1, :, 0:1] + 1.0)
    d1o[...] = d1
    d2o[...] = d2
    g1o[...] = d1 * jnp.dot(x1[...], w1[...], preferred_element_type=jnp.float32)
    g2o[...] = d2 * jnp.dot(x2[...], w2[...], preferred_element_type=jnp.float32)


def _tc_conv2_body(g2, pa, d2, b2, wog, g1, pb, d1, b1, l2o, gogo, l1ao):
    l2 = jnp.maximum(d2[...] * (g2[...] + pa[0] + pa[1]) + b2[...], 0.0)
    l2o[...] = l2
    gogo[...] = d2[...] * jnp.dot(l2, wog[...], preferred_element_type=jnp.float32)
    l1ao[...] = jnp.maximum(d1[...] * (g1[...] + pb[0] + pb[1]) + b1[...], 0.0)


def _tc_mid_body(l1a, q, gog, pc, d2, bog, l1o, f2o):
    l1o[...] = l1a[...] + q[0] + q[1]
    f2o[...] = jnp.maximum(d2[...] * (gog[...] + pc[0] + pc[1]) + bog[...], 0.0)


def _tc_emb_body(f2, r, wp1, d2, gp1o):
    xe = f2[...] + r[0] + r[1]
    gp1o[...] = d2[...] * jnp.dot(xe, wp1[...], preferred_element_type=jnp.float32)


def _tc_hid_body(gp1, pd, d2, bp1, gho):
    h = jnp.maximum(d2[...] * (gp1[...] + pd[0] + pd[1]) + bp1[...], 0.0)
    gho[...] = d2[...] * h


def _tc_out_body(gh, pe, d2, wp2, bp2, outo):
    y = d2[...] * (gh[...] + pe[0] + pe[1])
    outo[...] = jnp.dot(y, wp2[...], preferred_element_type=jnp.float32) + bp2[...]


def kernel(x_0, x_1, x_2, edge_index_0, edge_index_1, edge_index_2,
           layer_edge_index_0, layer_edge_index_1, layer_edge_index_2,
           W_lg_0, b_lg_0, W_lg_1, b_lg_1, W_lg_2, b_lg_2,
           W_og_0, b_og_0, W_og_1, b_og_1, W_og_2, b_og_2,
           W_p1, b_p1, W_p2, b_p2):
    ei1 = edge_index_1.astype(jnp.int32)
    ei2 = edge_index_2.astype(jnp.int32)
    lei2 = layer_edge_index_2.astype(jnp.int32)

    e_pad = _pad_len(ei2.shape[1])
    el_pad = _pad_len(lei2.shape[1])

    src1 = _pad_gather_idx(ei1[0], e_pad)
    dst1 = _pad_scatter_idx(ei1[1], e_pad)
    src2 = _pad_gather_idx(ei2[0], e_pad)
    dst2 = _pad_scatter_idx(ei2[1], e_pad)
    lg_in = _pad_gather_idx(lei2[0], el_pad)
    ls_in = _pad_scatter_idx(lei2[1], el_pad)
    lg_out = _pad_gather_idx(lei2[1], el_pad)
    ls_out = _pad_scatter_idx(lei2[0], el_pad)

    dsts = jnp.stack([
        _pad_scatter_idx(ei1[1], e_pad),
        _pad_scatter_idx(ei2[1], e_pad),
    ])

    zrows32 = jnp.zeros((ZR, F), jnp.float32)
    zrows16 = jnp.zeros((ZR, 16), jnp.float32)
    ones128 = jnp.ones((SUB, 16), jnp.float32)

    prop_e = _make_prop(e_pad)
    prop_l = _make_prop(el_pad)
    deg = _make_deg(e_pad)

    cnt = deg(dsts, ones128, zrows16)

    d1, d2, g1, g2 = _tc_call(
        _tc_prep_body,
        [_part_spec(16), _row_spec(F), _row_spec(F), _full_spec((F, F)),
         _full_spec((F, F))],
        4, (1, 1, F, F),
    )(cnt, x_1, x_2, W_lg_1, W_lg_2)

    pa = prop_e(g2, src2, dst2, zrows32)
    pb = prop_e(g1, src1, dst1, zrows32)

    blg1 = b_lg_1.reshape(1, F)
    blg2 = b_lg_2.reshape(1, F)
    bog2 = b_og_2.reshape(1, F)
    bp1 = b_p1.reshape(1, F)

    l2, gog, l1a = _tc_call(
        _tc_conv2_body,
        [_row_spec(F), _part_spec(F), _row_spec(1), _full_spec((1, F)),
         _full_spec((F, F)), _row_spec(F), _part_spec(F), _row_spec(1),
         _full_spec((1, F))],
        3, (F, F, F),
    )(g2, pa, d2, blg2, W_og_2, g1, pb, d1, blg1)

    pc = prop_e(gog, src2, dst2, zrows32)
    q = prop_l(l2, lg_out, ls_out, zrows32)

    l1, f2 = _tc_call(
        _tc_mid_body,
        [_row_spec(F), _part_spec(F), _row_spec(F), _part_spec(F),
         _row_spec(1), _full_spec((1, F))],
        2, (F, F),
    )(l1a, q, gog, pc, d2, bog2)

    r = prop_l(l1, lg_in, ls_in, zrows32)

    gp1, = _tc_call(
        _tc_emb_body,
        [_row_spec(F), _part_spec(F), _full_spec((F, F)), _row_spec(1)],
        1, (F,),
    )(f2, r, W_p1, d2)

    pd = prop_e(gp1, src2, dst2, zrows32)

    gh, = _tc_call(
        _tc_hid_body,
        [_row_spec(F), _part_spec(F), _row_spec(1), _full_spec((1, F))],
        1, (F,),
    )(gp1, pd, d2, bp1)

    pe = prop_e(gh, src2, dst2, zrows32)

    out, = _tc_call(
        _tc_out_body,
        [_row_spec(F), _part_spec(F), _row_spec(1), _full_spec((F, 2)),
         _full_spec((1, 2))],
        1, (2,),
    )(gh, pe, d2, W_p2, b_p2.reshape(1, 2))

    return out
